# Initial kernel scaffold; baseline (speedup 1.0000x reference)
import functools

import jax
import jax.numpy as jnp
from jax import lax
from jax.experimental import pallas as pl
from jax.experimental.pallas import tpu as pltpu
from jax.experimental.pallas import tpu_sc as plsc

N = 10000
FD = 128
K = 16
NP = 10240
RB = 128
G = NP // 128
CW = 512
MASKV = 1.0e37
PICKV = 1.0e38
IBIG = 2 ** 30
MAX_ROUNDS = 20

NC, NS = 2, 16
NW = NC * NS
EW = NP // NW
CE = 8
NGC = EW // CE
CS = 64
NSC = EW // CS
SLAB = NP // NS
CNTW = 16

_F32 = jnp.float32
_I32 = jnp.int32


def _knn_body(xb_ref, xt_ref, out_ref, d_ref):
    xb = xb_ref[...]
    sqb = jnp.sum(xb * xb, axis=1, keepdims=True)

    for c in range(NP // CW):
        xtc = xt_ref[:, pl.ds(c * CW, CW)]
        mm = lax.dot_general(xb, xtc, (((1,), (0,)), ((), ())),
                             precision=lax.Precision.HIGHEST,
                             preferred_element_type=_F32)
        sqc = jnp.sum(xtc * xtc, axis=0, keepdims=True)
        dis = sqb + sqc - 2.0 * mm
        colio = lax.broadcasted_iota(_I32, (1, CW), 1) + c * CW
        d_ref[:, pl.ds(c * CW, CW)] = jnp.where(colio >= N, MASKV, dis)

    lane = lax.broadcasted_iota(_I32, (RB, 128), 1)
    tlane = lax.broadcasted_iota(_I32, (RB, K), 1)

    def scan_pass(pcons, pgi):
        def gbody(g, carry):
            m, gi = carry
            blk = d_ref[:, pl.ds(g * 128, 128)]
            hit = jnp.logical_and(pcons, pgi == g)
            blk = jnp.where(hit, MASKV, blk)
            d_ref[:, pl.ds(g * 128, 128)] = blk
            better = blk < m
            gi = jnp.where(better, g, gi)
            m = jnp.where(better, blk, m)
            return m, gi
        m0 = jnp.full((RB, 128), PICKV, _F32)
        gi0 = jnp.zeros((RB, 128), _I32)
        return lax.fori_loop(0, G, gbody, (m0, gi0))

    def cond(carry):
        k, done = carry[0], carry[1]
        return jnp.logical_and(k < MAX_ROUNDS, jnp.logical_not(done))

    def body(carry):
        k, _, pcons, pgi, cv, ci = carry
        m, gi = scan_pass(pcons, pgi)
        cidx = gi * 128 + lane

        rv = jnp.min(m, axis=1, keepdims=True)
        ri = jnp.min(jnp.where(m == rv, cidx, IBIG), axis=1, keepdims=True)
        c16v = jnp.max(cv, axis=1, keepdims=True)
        c16i = jnp.max(jnp.where(cv == c16v, ci, -1), axis=1, keepdims=True)
        cert = jnp.logical_or(
            c16v < rv, jnp.logical_and(c16v == rv, c16i < ri))
        done = jnp.all(cert)

        ncv = jnp.full((RB, K), PICKV, _F32)
        nci = jnp.full((RB, K), IBIG, _I32)
        for t in range(K):
            av = jnp.min(m, axis=1, keepdims=True)
            ai = jnp.min(jnp.where(m == av, cidx, IBIG), axis=1, keepdims=True)
            bv = jnp.min(cv, axis=1, keepdims=True)
            bi = jnp.min(jnp.where(cv == bv, ci, IBIG), axis=1, keepdims=True)
            pa = jnp.logical_or(av < bv, jnp.logical_and(av == bv, ai < bi))
            v = jnp.where(pa, av, bv)
            i = jnp.where(pa, ai, bi)
            m = jnp.where(jnp.logical_and(m == v, cidx == i), PICKV, m)
            cv = jnp.where(jnp.logical_and(cv == v, ci == i), PICKV, cv)
            ncv = jnp.where(tlane == t, v, ncv)
            nci = jnp.where(tlane == t, i, nci)

        consumed = m == PICKV
        return k + 1, done, consumed, gi, ncv, nci

    init = (jnp.int32(0), jnp.bool_(False),
            jnp.zeros((RB, 128), jnp.bool_), jnp.zeros((RB, 128), _I32),
            jnp.full((RB, K), PICKV, _F32), jnp.full((RB, K), IBIG, _I32))
    carry = lax.while_loop(cond, body, init)
    nn = carry[5]
    out_ref[:, pl.ds(0, K)] = nn
    out_ref[:, pl.ds(K, 128 - K)] = jnp.zeros((RB, 128 - K), _I32)


_knn_call = pl.pallas_call(
    _knn_body,
    grid=(NP // RB,),
    in_specs=[pl.BlockSpec((RB, FD), lambda i: (i, 0)),
              pl.BlockSpec((FD, NP), lambda i: (0, 0))],
    out_specs=pl.BlockSpec((RB, 128), lambda i: (i, 0)),
    out_shape=jax.ShapeDtypeStruct((NP, 128), _I32),
    scratch_shapes=[pltpu.VMEM((RB, NP), _F32)],
)


def _mm_body(x_ref, w_ref, o_ref):
    o_ref[...] = lax.dot_general(
        x_ref[...], w_ref[...], (((1,), (0,)), ((), ())),
        precision=lax.Precision.HIGHEST, preferred_element_type=_F32)


_mm_call = pl.pallas_call(
    _mm_body,
    grid=(NP // 512,),
    in_specs=[pl.BlockSpec((512, FD), lambda i: (i, 0)),
              pl.BlockSpec((FD, FD), lambda i: (0, 0))],
    out_specs=pl.BlockSpec((512, FD), lambda i: (i, 0)),
    out_shape=jax.ShapeDtypeStruct((NP, FD), _F32),
)


def _combine(p0, p1, c0, c1, b):
    s = p0 + p1
    dn = jnp.maximum(c0[:, 0:1] + c1[:, 0:1], 1.0)
    z = s / dn + b
    return jnp.where(z >= 0.0, z, 0.01 * z)


def _comb_mm_body(p0_ref, p1_ref, c0_ref, c1_ref, b_ref, w_ref, o_ref):
    z = _combine(p0_ref[...], p1_ref[...], c0_ref[...], c1_ref[...], b_ref[...])
    o_ref[...] = lax.dot_general(
        z, w_ref[...], (((1,), (0,)), ((), ())),
        precision=lax.Precision.HIGHEST, preferred_element_type=_F32)


def _comb_body(p0_ref, p1_ref, c0_ref, c1_ref, b_ref, o_ref):
    o_ref[...] = _combine(
        p0_ref[...], p1_ref[...], c0_ref[...], c1_ref[...], b_ref[...])


_comb_specs = [pl.BlockSpec((512, FD), lambda i: (i, 0)),
               pl.BlockSpec((512, FD), lambda i: (i, 0)),
               pl.BlockSpec((512, CNTW), lambda i: (i, 0)),
               pl.BlockSpec((512, CNTW), lambda i: (i, 0)),
               pl.BlockSpec((1, FD), lambda i: (0, 0))]

_comb_mm_call = pl.pallas_call(
    _comb_mm_body,
    grid=(NP // 512,),
    in_specs=_comb_specs + [pl.BlockSpec((FD, FD), lambda i: (0, 0))],
    out_specs=pl.BlockSpec((512, FD), lambda i: (i, 0)),
    out_shape=jax.ShapeDtypeStruct((NP, FD), _F32),
)

_comb_call = pl.pallas_call(
    _comb_body,
    grid=(NP // 512,),
    in_specs=_comb_specs,
    out_specs=pl.BlockSpec((512, FD), lambda i: (i, 0)),
    out_shape=jax.ShapeDtypeStruct((NP, FD), _F32),
)


def _sc_layer_body(y_hbm, ga_hbm, sct_hbm, zrow_hbm, zcnt_hbm,
                   p_hbm, cnt_hbm,
                   gidx_v, rows_v, ft_v, sidx_v, ones_v, acc_sh, accc_sh, sem):
    c = lax.axis_index("c")
    s = lax.axis_index("s")
    wid = c * NS + s
    base_e = wid * EW
    slab = s * SLAB

    pltpu.sync_copy(zrow_hbm, acc_sh.at[pl.ds(slab, SLAB)])
    pltpu.sync_copy(zcnt_hbm, accc_sh.at[pl.ds(slab, SLAB)])

    def ones_body(i, _):
        ones_v[i, :] = jnp.ones((CNTW,), _F32)
        return 0
    lax.fori_loop(0, CS, ones_body, 0)
    plsc.subcore_barrier()

    def gchunk(t, _):
        eb = base_e + t * CE
        pltpu.sync_copy(ga_hbm.at[pl.ds(eb * K, CE * K)], gidx_v)
        pltpu.async_copy(y_hbm.at[gidx_v], rows_v, sem).wait()

        def ebody(e, _):
            r0 = e * K
            for v in range(FD // 16):
                acc = rows_v[r0, pl.ds(v * 16, 16)]
                for j in range(1, K):
                    acc = acc + rows_v[r0 + j, pl.ds(v * 16, 16)]
                ft_v[t * CE + e, pl.ds(v * 16, 16)] = acc * (1.0 / K)
            return 0
        lax.fori_loop(0, CE, ebody, 0)
        return 0
    lax.fori_loop(0, NGC, gchunk, 0)

    def schunk(u, _):
        eb = base_e + u * CS

        def jbody(j, _):
            pltpu.sync_copy(sct_hbm.at[j, pl.ds(eb, CS)], sidx_v)
            pltpu.sync_copy(ft_v.at[pl.ds(u * CS, CS)],
                            acc_sh.at[sidx_v], add=True)
            pltpu.sync_copy(ones_v, accc_sh.at[sidx_v], add=True)
            return 0
        lax.fori_loop(0, K, jbody, 0)
        return 0
    lax.fori_loop(0, NSC, schunk, 0)
    plsc.subcore_barrier()

    pltpu.sync_copy(acc_sh.at[pl.ds(slab, SLAB)],
                    p_hbm.at[c, pl.ds(slab, SLAB)])
    pltpu.sync_copy(accc_sh.at[pl.ds(slab, SLAB)],
                    cnt_hbm.at[c, pl.ds(slab, SLAB)])


_sc_layer = functools.partial(
    pl.kernel,
    out_type=(jax.ShapeDtypeStruct((NC, NP, FD), _F32),
              jax.ShapeDtypeStruct((NC, NP, CNTW), _F32)),
    mesh=plsc.VectorSubcoreMesh(core_axis_name="c", subcore_axis_name="s"),
    scratch_types=[
        pltpu.VMEM((CE * K,), _I32),
        pltpu.VMEM((CE * K, FD), _F32),
        pltpu.VMEM((EW, FD), _F32),
        pltpu.VMEM((CS,), _I32),
        pltpu.VMEM((CS, CNTW), _F32),
        pltpu.VMEM_SHARED((NP, FD), _F32),
        pltpu.VMEM_SHARED((NP, CNTW), _F32),
        pltpu.SemaphoreType.DMA,
    ],
)(_sc_layer_body)


def kernel(x, W0, b0, W1, b1):
    xp = jnp.pad(x, ((0, NP - N), (0, 0)))
    xt = xp.T
    nn = _knn_call(xp, xt)[:N, :K]

    ga = jnp.pad(nn, ((0, NP - N), (0, 0))).reshape(-1)
    sct = jnp.pad(nn, ((0, NP - N), (0, 0)), constant_values=N).T
    sct = jnp.asarray(sct, _I32)
    zrow = jnp.zeros((SLAB, FD), _F32)
    zcnt = jnp.zeros((SLAB, CNTW), _F32)

    y = _mm_call(xp, W0)
    p, cnt = _sc_layer(y, ga, sct, zrow, zcnt)
    y = _comb_mm_call(p[0], p[1], cnt[0], cnt[1], b0[None, :], W1)
    p, cnt2 = _sc_layer(y, ga, sct, zrow, zcnt)
    del cnt2
    out = _comb_call(p[0], p[1], cnt[0], cnt[1], b1[None, :])
    return out[:N]


# speedup vs baseline: 4.0004x; 4.0004x over previous
import functools

import jax
import jax.numpy as jnp
from jax import lax
from jax.experimental import pallas as pl
from jax.experimental.pallas import tpu as pltpu
from jax.experimental.pallas import tpu_sc as plsc

N = 10000
FD = 128
K = 16
NP = 10240
RB = 128
G = NP // 128
CW = 512
MASKV = 1.0e37
PICKV = 1.0e38
IBIG = 2 ** 30
MAX_ROUNDS = 20

NC, NS = 2, 16
HF = FD // 2
EW = NP // NS
CE = 8
CS = 128
NSS = EW // CS
NGS = CS // CE
NGC = EW // CE
PUB = 64
SLAB = NP // NS
CNTW = 16

_F32 = jnp.float32
_I32 = jnp.int32


def _knn_body(xb_ref, xt_ref, out_ref, d_ref):
    xb = xb_ref[...]
    sqb = jnp.sum(xb * xb, axis=1, keepdims=True)

    for c in range(NP // CW):
        xtc = xt_ref[:, pl.ds(c * CW, CW)]
        mm = lax.dot_general(xb, xtc, (((1,), (0,)), ((), ())),
                             precision=lax.Precision.DEFAULT,
                             preferred_element_type=_F32)
        sqc = jnp.sum(xtc * xtc, axis=0, keepdims=True)
        dis = sqb + sqc - 2.0 * mm
        colio = lax.broadcasted_iota(_I32, (1, CW), 1) + c * CW
        d_ref[:, pl.ds(c * CW, CW)] = jnp.where(colio >= N, MASKV, dis)

    lane = lax.broadcasted_iota(_I32, (RB, 128), 1)
    tlane = lax.broadcasted_iota(_I32, (RB, K), 1)

    def scan_pass(pcons, pgi):
        def gbody(g, carry):
            m, gi = carry
            blk = d_ref[:, pl.ds(g * 128, 128)]
            hit = jnp.logical_and(pcons, pgi == g)
            blk = jnp.where(hit, MASKV, blk)
            d_ref[:, pl.ds(g * 128, 128)] = blk
            better = blk < m
            gi = jnp.where(better, g, gi)
            m = jnp.where(better, blk, m)
            return m, gi
        m0 = jnp.full((RB, 128), PICKV, _F32)
        gi0 = jnp.zeros((RB, 128), _I32)
        return lax.fori_loop(0, G, gbody, (m0, gi0))

    def cond(carry):
        k, done = carry[0], carry[1]
        return jnp.logical_and(k < MAX_ROUNDS, jnp.logical_not(done))

    def body(carry):
        k, _, pconsi, pgi, cv, ci = carry
        m, gi = scan_pass(pconsi == 1, pgi)
        cidx = gi * 128 + lane

        rv = jnp.min(m, axis=1, keepdims=True)
        ri = jnp.min(jnp.where(m == rv, cidx, IBIG), axis=1, keepdims=True)
        c16v = jnp.max(cv, axis=1, keepdims=True)
        c16i = jnp.max(jnp.where(cv == c16v, ci, -1), axis=1, keepdims=True)
        cert = jnp.logical_or(
            c16v < rv, jnp.logical_and(c16v == rv, c16i < ri))
        done = jnp.all(cert)

        ncv = jnp.full((RB, K), PICKV, _F32)
        nci = jnp.full((RB, K), IBIG, _I32)
        for t in range(K):
            av = jnp.min(m, axis=1, keepdims=True)
            ai = jnp.min(jnp.where(m == av, cidx, IBIG), axis=1, keepdims=True)
            bv = jnp.min(cv, axis=1, keepdims=True)
            bi = jnp.min(jnp.where(cv == bv, ci, IBIG), axis=1, keepdims=True)
            pa = jnp.logical_or(av < bv, jnp.logical_and(av == bv, ai < bi))
            v = jnp.where(pa, av, bv)
            i = jnp.where(pa, ai, bi)
            m = jnp.where(jnp.logical_and(m == v, cidx == i), PICKV, m)
            cv = jnp.where(jnp.logical_and(cv == v, ci == i), PICKV, cv)
            ncv = jnp.where(tlane == t, v, ncv)
            nci = jnp.where(tlane == t, i, nci)

        consumed = jnp.where(m == PICKV, 1, 0)
        return k + 1, done, consumed, gi, ncv, nci

    init = (jnp.int32(0), jnp.bool_(False),
            jnp.zeros((RB, 128), _I32), jnp.zeros((RB, 128), _I32),
            jnp.full((RB, K), PICKV, _F32), jnp.full((RB, K), IBIG, _I32))
    carry = lax.while_loop(cond, body, init)
    nn = carry[5]
    out_ref[:, pl.ds(0, K)] = nn
    out_ref[:, pl.ds(K, 128 - K)] = jnp.zeros((RB, 128 - K), _I32)


_knn_call = pl.pallas_call(
    _knn_body,
    grid=(NP // RB,),
    in_specs=[pl.BlockSpec((RB, FD), lambda i: (i, 0)),
              pl.BlockSpec((FD, NP), lambda i: (0, 0))],
    out_specs=pl.BlockSpec((RB, 128), lambda i: (i, 0)),
    out_shape=jax.ShapeDtypeStruct((NP, 128), _I32),
    scratch_shapes=[pltpu.VMEM((RB, NP), _F32)],
)


def _mm_body(x_ref, w_ref, o_ref):
    o_ref[...] = lax.dot_general(
        x_ref[...], w_ref[...], (((1,), (0,)), ((), ())),
        precision=lax.Precision.HIGHEST, preferred_element_type=_F32)


_mm_call = pl.pallas_call(
    _mm_body,
    grid=(NP // 512,),
    in_specs=[pl.BlockSpec((512, FD), lambda i: (i, 0)),
              pl.BlockSpec((FD, FD), lambda i: (0, 0))],
    out_specs=pl.BlockSpec((512, FD), lambda i: (i, 0)),
    out_shape=jax.ShapeDtypeStruct((NP, FD), _F32),
)


def _comb_body(p0_ref, p1_ref, cn_ref, b_ref, o_ref):
    s = jnp.concatenate([p0_ref[...], p1_ref[...]], axis=1)
    dn = jnp.maximum(cn_ref[...], 1.0)
    z = s / dn + b_ref[...]
    o_ref[...] = jnp.where(z >= 0.0, z, 0.01 * z)


_comb_specs = [pl.BlockSpec((512, HF), lambda i: (i, 0)),
               pl.BlockSpec((512, HF), lambda i: (i, 0)),
               pl.BlockSpec((512, 1), lambda i: (i, 0)),
               pl.BlockSpec((1, FD), lambda i: (0, 0))]

_comb_call = pl.pallas_call(
    _comb_body,
    grid=(NP // 512,),
    in_specs=_comb_specs,
    out_specs=pl.BlockSpec((512, FD), lambda i: (i, 0)),
    out_shape=jax.ShapeDtypeStruct((NP, FD), _F32),
)


def _sc_layer_body(y_hbm, ga_hbm, sct_hbm, zrow_hbm, zcnt_hbm, p_hbm,
                   cnt_hbm, gidx_all, sidx_all, rows_v, ftc_v, ones_v,
                   acc_sh, accc_sh, sem):
    c = lax.axis_index("c")
    s = lax.axis_index("s")
    slab = s * SLAB
    lane0 = c * HF

    pltpu.sync_copy(ga_hbm.at[s], gidx_all)
    pltpu.sync_copy(sct_hbm.at[s], sidx_all)

    pltpu.sync_copy(zrow_hbm, acc_sh.at[pl.ds(slab, SLAB)])
    pltpu.sync_copy(zcnt_hbm, accc_sh.at[pl.ds(slab, SLAB)])

    def ones_body(i, _):
        ones_v[i, :] = jnp.ones((CNTW,), _F32)
        return 0
    lax.fori_loop(0, CS, ones_body, 0)
    plsc.subcore_barrier()

    def schunk(u, _):
        def gsub(g, _):
            t = u * NGS + g
            pltpu.async_copy(y_hbm.at[gidx_all.at[t]], rows_v, sem).wait()

            def ebody(e, _):
                r0 = e * K
                for v in range(HF // 16):
                    src = lane0 + v * 16
                    acc = rows_v[r0, pl.ds(src, 16)]
                    for j in range(1, K):
                        acc = acc + rows_v[r0 + j, pl.ds(src, 16)]
                    ftc_v[g * CE + e, pl.ds(v * 16, 16)] = acc * (1.0 / K)
                return 0
            lax.fori_loop(0, CE, ebody, 0)
            return 0
        lax.fori_loop(0, NGS, gsub, 0)

        def jbody(j, _):
            m = j * NSS + u
            pltpu.sync_copy(ftc_v, acc_sh.at[sidx_all.at[m]], add=True)

            @pl.when(c == 0)
            def _():
                pltpu.sync_copy(ones_v, accc_sh.at[sidx_all.at[m]], add=True)
            return 0
        lax.fori_loop(0, K, jbody, 0)
        return 0
    lax.fori_loop(0, NSS, schunk, 0)
    plsc.subcore_barrier()

    def pub(i, _):
        r = slab + i * PUB
        pltpu.sync_copy(acc_sh.at[pl.ds(r, PUB)], p_hbm.at[c, pl.ds(r, PUB)])
        return 0
    lax.fori_loop(0, SLAB // PUB, pub, 0)

    @pl.when(c == 0)
    def _():
        pltpu.sync_copy(accc_sh.at[pl.ds(slab, SLAB)],
                        cnt_hbm.at[pl.ds(slab, SLAB)])


@functools.cache
def _get_sc_layer():
    mesh = plsc.VectorSubcoreMesh(core_axis_name="c", subcore_axis_name="s",
                                  num_cores=NC, num_subcores=NS)
    return functools.partial(
        pl.kernel,
        out_type=(jax.ShapeDtypeStruct((NC, NP, HF), _F32),
                  jax.ShapeDtypeStruct((NP, CNTW), _F32)),
        mesh=mesh,
        compiler_params=pltpu.CompilerParams(use_tc_tiling_on_sc=False),
        scratch_types=[
            pltpu.VMEM((NGC, CE * K), _I32),
            pltpu.VMEM((K * NSS, CS), _I32),
            pltpu.VMEM((CE * K, FD), _F32),
            pltpu.VMEM((CS, HF), _F32),
            pltpu.VMEM((CS, CNTW), _F32),
            pltpu.VMEM_SHARED((NP, HF), _F32),
            pltpu.VMEM_SHARED((NP, CNTW), _F32),
            pltpu.SemaphoreType.DMA,
        ],
    )(_sc_layer_body)


def kernel(x, W0, b0, W1, b1):
    xp = jnp.pad(x, ((0, NP - N), (0, 0)))
    xt = xp.T
    nn = _knn_call(xp, xt)[:N, :K]

    ga = jnp.pad(nn, ((0, NP - N), (0, 0))).reshape(NS, NGC, CE * K)
    sct = jnp.pad(nn, ((0, NP - N), (0, 0)), constant_values=N).T
    sct = jnp.asarray(sct, _I32).reshape(K, NS, NSS, CS)
    sct = sct.transpose(1, 0, 2, 3).reshape(NS, K * NSS, CS)
    zrow = jnp.zeros((SLAB, HF), _F32)
    zcnt = jnp.zeros((SLAB, CNTW), _F32)

    sc_layer = _get_sc_layer()
    Ws = jnp.stack([W0, W1])
    bs = jnp.stack([b0, b1])

    def step(z, wb):
        W, b = wb
        yf = _mm_call(z, W)
        p, cnt = sc_layer(yf, ga, sct, zrow, zcnt)
        z2 = _comb_call(p[0], p[1], cnt[:, 0:1], b[None, :])
        return z2, 0

    z, _ = lax.scan(step, xp, (Ws, bs))
    return z[:N]
